# Initial kernel scaffold; baseline (speedup 1.0000x reference)
import jax
import jax.numpy as jnp
from jax.experimental import pallas as pl


def kernel(x, table):
    raise NotImplementedError("write your pallas kernel here")


# speedup vs baseline: 1.0129x; 1.0129x over previous
import functools

import jax
import jax.numpy as jnp
from jax import lax
from jax.experimental import pallas as pl
from jax.experimental.pallas import tpu as pltpu
from jax.experimental.pallas import tpu_sc as plsc

_NUM_CORES = 2
_NUM_SUBCORES = 16
_NUM_WORKERS = _NUM_CORES * _NUM_SUBCORES
_CHUNK = 512
_BATCH_BLOCK = 64


def _sc_gather(table4, idx4):
    num_rows = idx4.shape[0]
    rows_per_worker = num_rows // _NUM_WORKERS
    n_chunks = rows_per_worker // _CHUNK
    mesh = plsc.VectorSubcoreMesh(core_axis_name="c", subcore_axis_name="s")

    @functools.partial(
        pl.kernel,
        mesh=mesh,
        out_type=jax.ShapeDtypeStruct((num_rows, 128), jnp.float32),
        scratch_types=[
            pltpu.VMEM((_CHUNK,), jnp.int32),
            pltpu.VMEM((_CHUNK, 128), jnp.float32),
            pltpu.SemaphoreType.DMA,
        ],
    )
    def gather_kernel(table_hbm, idx_hbm, out_hbm, idx_v, lines_v, sem):
        wid = lax.axis_index("s") * _NUM_CORES + lax.axis_index("c")
        base = wid * rows_per_worker

        @pl.loop(0, n_chunks)
        def _(c):
            off = base + c * _CHUNK
            pltpu.sync_copy(idx_hbm.at[pl.ds(off, _CHUNK)], idx_v)
            pltpu.async_copy(table_hbm.at[idx_v], lines_v, sem).wait()
            pltpu.sync_copy(lines_v, out_hbm.at[pl.ds(off, _CHUNK)])

    return gather_kernel(table4, idx4)


def _tc_select(lines, rem):
    batch, fields = rem.shape
    dim = 32
    n_blocks = batch // _BATCH_BLOCK

    def select_kernel(lines_ref, rem_ref, out_ref):
        a = lines_ref[...].reshape(_BATCH_BLOCK, fields, 128)
        r = rem_ref[...].reshape(_BATCH_BLOCK, fields, 1)
        w = jnp.where(
            r < 2,
            jnp.where(r == 0, a[:, :, 0:dim], a[:, :, dim : 2 * dim]),
            jnp.where(r == 2, a[:, :, 2 * dim : 3 * dim], a[:, :, 3 * dim :]),
        )
        out_ref[...] = w

    return pl.pallas_call(
        select_kernel,
        grid=(n_blocks,),
        in_specs=[
            pl.BlockSpec((_BATCH_BLOCK * fields, 128), lambda i: (i, 0)),
            pl.BlockSpec((_BATCH_BLOCK, fields), lambda i: (i, 0)),
        ],
        out_specs=pl.BlockSpec((_BATCH_BLOCK, fields, dim), lambda i: (i, 0, 0)),
        out_shape=jax.ShapeDtypeStruct((batch, fields, dim), jnp.float32),
    )(lines, rem)


def kernel(x, table):
    batch, fields = x.shape
    flat = x.reshape(batch * fields).astype(jnp.int32)
    idx4 = flat // 4
    rem = (x % 4).astype(jnp.int32)
    table4 = table.reshape(table.shape[0] // 4, 128)
    lines = _sc_gather(table4, idx4)
    return _tc_select(lines, rem)
